# baseline (speedup 1.0000x reference)
import functools

import jax
import jax.numpy as jnp
from jax import lax
from jax.experimental import pallas as pl
from jax.experimental.pallas import tpu as pltpu
from jax.experimental.pallas import tpu_sc as plsc

_N = 10000
_E = 320000
_FIN = 128
_H = 64
_OUT = 32
_G = 64

_NC, _NS = 2, 16
_NW = _NC * _NS
_CH = 128
_NCH = 80
_EPW = _NCH * _CH
_NBUF = 4
_ACC = 10240
_ZCH = _ACC // _NS
_BN = 400

_DOT = dict(preferred_element_type=jnp.float32, precision=lax.Precision.HIGHEST)


def _zero_rows(buf, nrows, width):
    @pl.loop(0, nrows)
    def _(i):
        for j0 in range(0, width, 16):
            buf[i, pl.ds(j0, 16)] = jnp.zeros((16,), jnp.float32)


@functools.lru_cache(maxsize=None)
def _sc_degree_kernel():
    mesh = plsc.VectorSubcoreMesh(core_axis_name="c", subcore_axis_name="s")
    return pl.kernel(
        _sc_degree,
        out_type=jax.ShapeDtypeStruct((_NC, _ACC, 16), jnp.float32),
        mesh=mesh,
        compiler_params=pltpu.CompilerParams(use_tc_tiling_on_sc=False),
        scratch_types=[
            pltpu.VMEM((_NCH, _CH), jnp.int32),
            pltpu.VMEM((_CH, 16), jnp.float32),
            pltpu.VMEM_SHARED((_ACC, 16), jnp.float32),
        ],
    )


@functools.lru_cache(maxsize=None)
def _sc_aggregate_kernel():
    mesh = plsc.VectorSubcoreMesh(core_axis_name="c", subcore_axis_name="s")
    return pl.kernel(
        _sc_aggregate,
        out_type=jax.ShapeDtypeStruct((_NC, _ACC, _H), jnp.float32),
        mesh=mesh,
        compiler_params=pltpu.CompilerParams(use_tc_tiling_on_sc=False),
        scratch_types=[
            pltpu.VMEM((_NCH, _CH), jnp.int32),
            pltpu.VMEM((_NCH, _CH), jnp.int32),
            [pltpu.VMEM((_CH, _H), jnp.float32)] * _NBUF,
            pltpu.VMEM_SHARED((_ACC, _H), jnp.float32),
            [pltpu.SemaphoreType.DMA] * _NBUF,
        ],
    )


def _sc_degree(dst_hbm, out_hbm, didx, rbuf, acc):
    c = lax.axis_index("c")
    s = lax.axis_index("s")
    wid = s * _NC + c
    _zero_rows(rbuf, _CH, 16)

    @pl.loop(0, _ZCH // _CH)
    def _(k):
        pltpu.sync_copy(rbuf, acc.at[pl.ds(s * _ZCH + k * _CH, _CH)])

    lanes = lax.iota(jnp.int32, 16)
    one0 = jnp.where(lanes == 0, jnp.float32(1.0), jnp.float32(0.0))

    @pl.loop(0, _CH)
    def _(i):
        rbuf[i, pl.ds(0, 16)] = one0

    pltpu.sync_copy(dst_hbm.at[wid], didx)
    plsc.subcore_barrier()

    @pl.loop(0, _NCH)
    def _(j):
        pltpu.sync_copy(rbuf, acc.at[didx.at[j]], add=True)

    plsc.subcore_barrier()
    pltpu.sync_copy(acc.at[pl.ds(s * _ZCH, _ZCH)],
                    out_hbm.at[c, pl.ds(s * _ZCH, _ZCH)])


def _sc_aggregate(y_hbm, src_hbm, dst_hbm, out_hbm, sidx, didx, rows, acc,
                  sems):
    c = lax.axis_index("c")
    s = lax.axis_index("s")
    wid = s * _NC + c
    _zero_rows(rows[0], _CH, _H)

    @pl.loop(0, _ZCH // _CH)
    def _(k):
        pltpu.sync_copy(rows[0], acc.at[pl.ds(s * _ZCH + k * _CH, _CH)])

    pltpu.sync_copy(src_hbm.at[wid], sidx)
    pltpu.sync_copy(dst_hbm.at[wid], didx)
    plsc.subcore_barrier()

    @pl.loop(0, _NCH, step=_NBUF)
    def _(j):
        cps = [pltpu.async_copy(y_hbm.at[sidx.at[j + b]], rows[b], sems[b])
               for b in range(_NBUF)]
        for b in range(_NBUF):
            cps[b].wait()
            pltpu.sync_copy(rows[b], acc.at[didx.at[j + b]], add=True)

    plsc.subcore_barrier()
    pltpu.sync_copy(acc.at[pl.ds(s * _ZCH, _ZCH)],
                    out_hbm.at[c, pl.ds(s * _ZCH, _ZCH)])


def _tc1_body(d0, d1, x, w, y_out, dinv_out):
    deg = d0[:, 0:1] + d1[:, 0:1] + 1.0
    dv = 1.0 / jnp.sqrt(deg)
    dinv_out[...] = dv
    xw = lax.dot_general(x[...], w[...], (((1,), (0,)), ((), ())), **_DOT)
    y_out[...] = dv * xw


_tc1 = pl.pallas_call(
    _tc1_body,
    grid=(_N // _BN,),
    in_specs=[
        pl.BlockSpec((_BN, 16), lambda i: (i, 0)),
        pl.BlockSpec((_BN, 16), lambda i: (i, 0)),
        pl.BlockSpec((_BN, _FIN), lambda i: (i, 0)),
        pl.BlockSpec((_FIN, _H), lambda i: (0, 0)),
    ],
    out_specs=[
        pl.BlockSpec((_BN, _H), lambda i: (i, 0)),
        pl.BlockSpec((_BN, 1), lambda i: (i, 0)),
    ],
    out_shape=[
        jax.ShapeDtypeStruct((_N, _H), jnp.float32),
        jax.ShapeDtypeStruct((_N, 1), jnp.float32),
    ],
)


def _tc2_body(p0, p1, y, dinv, b, w, yn):
    dv = dinv[...]
    h = jnp.maximum(dv * (p0[...] + p1[...] + y[...]) + b[...], 0.0)
    yn[...] = dv * lax.dot_general(h, w[...], (((1,), (0,)), ((), ())), **_DOT)


_tc2 = pl.pallas_call(
    _tc2_body,
    grid=(_N // _BN,),
    in_specs=[
        pl.BlockSpec((_BN, _H), lambda i: (i, 0)),
        pl.BlockSpec((_BN, _H), lambda i: (i, 0)),
        pl.BlockSpec((_BN, _H), lambda i: (i, 0)),
        pl.BlockSpec((_BN, 1), lambda i: (i, 0)),
        pl.BlockSpec((1, _H), lambda i: (0, 0)),
        pl.BlockSpec((_H, _H), lambda i: (0, 0)),
    ],
    out_specs=pl.BlockSpec((_BN, _H), lambda i: (i, 0)),
    out_shape=jax.ShapeDtypeStruct((_N, _H), jnp.float32),
)


def _tc3_body(p0, p1, y3, dinv, b3, bat, pw1, pb1, pw2, pb2, zout, pooled, cnt):
    i = pl.program_id(0)

    @pl.when(i == 0)
    def _():
        pooled[...] = jnp.zeros_like(pooled)
        cnt[...] = jnp.zeros_like(cnt)

    dv = dinv[...]
    h3 = dv * (p0[...] + p1[...] + y3[...]) + b3[...]
    gid = lax.broadcasted_iota(jnp.int32, (_BN, _G), 1)
    oh = (bat[...] == gid).astype(jnp.float32)
    pooled[...] += lax.dot_general(oh, h3, (((0,), (0,)), ((), ())), **_DOT)
    cnt[...] += lax.dot_general(oh, jnp.ones((_BN, 1), jnp.float32),
                                (((0,), (0,)), ((), ())), **_DOT)

    @pl.when(i == _N // _BN - 1)
    def _():
        hm = pooled[...] / jnp.maximum(cnt[...], 1.0)
        z = jnp.maximum(lax.dot_general(hm, pw1[...], (((1,), (0,)), ((), ())),
                                        **_DOT) + pb1[...], 0.0)
        z = lax.dot_general(z, pw2[...], (((1,), (0,)), ((), ())), **_DOT) + pb2[...]
        nrm = jnp.sqrt(jnp.sum(z * z, axis=1, keepdims=True))
        zout[...] = z / jnp.maximum(nrm, 1e-12)


_tc3 = pl.pallas_call(
    _tc3_body,
    grid=(_N // _BN,),
    in_specs=[
        pl.BlockSpec((_BN, _H), lambda i: (i, 0)),
        pl.BlockSpec((_BN, _H), lambda i: (i, 0)),
        pl.BlockSpec((_BN, _H), lambda i: (i, 0)),
        pl.BlockSpec((_BN, 1), lambda i: (i, 0)),
        pl.BlockSpec((1, _H), lambda i: (0, 0)),
        pl.BlockSpec((_BN, 1), lambda i: (i, 0)),
        pl.BlockSpec((_H, _H), lambda i: (0, 0)),
        pl.BlockSpec((1, _H), lambda i: (0, 0)),
        pl.BlockSpec((_H, _OUT), lambda i: (0, 0)),
        pl.BlockSpec((1, _OUT), lambda i: (0, 0)),
    ],
    out_specs=pl.BlockSpec((_G, _OUT), lambda i: (0, 0)),
    out_shape=jax.ShapeDtypeStruct((_G, _OUT), jnp.float32),
    scratch_shapes=[
        pltpu.VMEM((_G, _H), jnp.float32),
        pltpu.VMEM((_G, 1), jnp.float32),
    ],
)


def kernel(x, edge_index, batch, W1, b1, W2, b2, W3, b3, PW1, Pb1, PW2, Pb2):
    src = edge_index[0]
    dst = edge_index[1]
    pad = _NW * _EPW - _E
    src_p = jnp.concatenate([src, jnp.zeros((pad,), jnp.int32)]).reshape(_NW, _NCH, _CH)
    dst_p = jnp.concatenate([dst, jnp.full((pad,), _N, jnp.int32)]).reshape(_NW, _NCH, _CH)

    sc_deg = _sc_degree_kernel()
    sc_agg = _sc_aggregate_kernel()
    degp = sc_deg(dst_p)[:, :_N]
    y1, dinv = _tc1(degp[0], degp[1], x, W1)
    p = sc_agg(y1, src_p, dst_p)[:, :_N]
    y2 = _tc2(p[0], p[1], y1, dinv, b1.reshape(1, _H), W2)
    p = sc_agg(y2, src_p, dst_p)[:, :_N]
    y3 = _tc2(p[0], p[1], y2, dinv, b2.reshape(1, _H), W3)
    p = sc_agg(y3, src_p, dst_p)[:, :_N]
    return _tc3(p[0], p[1], y3, dinv, b3.reshape(1, _H), batch.reshape(_N, 1),
                PW1, Pb1.reshape(1, _H), PW2, Pb2.reshape(1, _OUT))


# speedup vs baseline: 1.7298x; 1.7298x over previous
import functools

import jax
import jax.numpy as jnp
from jax import lax
from jax.experimental import pallas as pl
from jax.experimental.pallas import tpu as pltpu
from jax.experimental.pallas import tpu_sc as plsc

_N = 10000
_E = 320000
_FIN = 128
_H = 64
_HH = _H // 2
_OUT = 32
_G = 64

_NC, _NS = 2, 16
_CH = 128
_NCH = 160
_EPW = _NCH * _CH
_EPAD = _NS * _EPW
_ACC = 10240
_ZCH = _ACC // _NS
_DCH = _NCH // _NC
_BN = 400

_DOT = dict(preferred_element_type=jnp.float32, precision=lax.Precision.HIGHEST)


def _zero_rows(buf, nrows, width):
    @pl.loop(0, nrows)
    def _(i):
        for j0 in range(0, width, 16):
            buf[i, pl.ds(j0, 16)] = jnp.zeros((16,), jnp.float32)


@functools.lru_cache(maxsize=None)
def _sc_degree_kernel():
    mesh = plsc.VectorSubcoreMesh(core_axis_name="c", subcore_axis_name="s")
    return pl.kernel(
        _sc_degree,
        out_type=jax.ShapeDtypeStruct((_NC, _ACC, 16), jnp.float32),
        mesh=mesh,
        compiler_params=pltpu.CompilerParams(use_tc_tiling_on_sc=False),
        scratch_types=[
            pltpu.VMEM((_DCH, _CH), jnp.int32),
            pltpu.VMEM((_CH, 16), jnp.float32),
            pltpu.VMEM_SHARED((_ACC, 16), jnp.float32),
        ],
    )


@functools.lru_cache(maxsize=None)
def _sc_aggregate_kernel():
    mesh = plsc.VectorSubcoreMesh(core_axis_name="c", subcore_axis_name="s")
    return pl.kernel(
        _sc_aggregate,
        out_type=jax.ShapeDtypeStruct((_NC, _ACC, _HH), jnp.float32),
        mesh=mesh,
        compiler_params=pltpu.CompilerParams(use_tc_tiling_on_sc=False),
        scratch_types=[
            pltpu.VMEM((_NCH, _CH), jnp.int32),
            pltpu.VMEM((_NCH, _CH), jnp.int32),
            pltpu.VMEM((_CH, _HH), jnp.float32),
            pltpu.VMEM_SHARED((_N, _HH), jnp.float32),
            pltpu.VMEM_SHARED((_ACC, _HH), jnp.float32),
        ],
    )


def _sc_degree(dst_hbm, out_hbm, didx, rbuf, acc):
    c = lax.axis_index("c")
    s = lax.axis_index("s")
    _zero_rows(rbuf, _CH, 16)

    @pl.loop(0, _ZCH // _CH)
    def _(k):
        pltpu.sync_copy(rbuf, acc.at[pl.ds(s * _ZCH + k * _CH, _CH)])

    lanes = lax.iota(jnp.int32, 16)
    one0 = jnp.where(lanes == 0, jnp.float32(1.0), jnp.float32(0.0))

    @pl.loop(0, _CH)
    def _(i):
        rbuf[i, pl.ds(0, 16)] = one0

    pltpu.sync_copy(dst_hbm.at[s, pl.ds(c * _DCH, _DCH)], didx)
    plsc.subcore_barrier()

    @pl.loop(0, _DCH)
    def _(j):
        pltpu.sync_copy(rbuf, acc.at[didx.at[j]], add=True)

    plsc.subcore_barrier()
    pltpu.sync_copy(acc.at[pl.ds(s * _ZCH, _ZCH)],
                    out_hbm.at[c, pl.ds(s * _ZCH, _ZCH)])


def _sc_aggregate(yl_hbm, yr_hbm, src_hbm, dst_hbm, out_hbm, sidx, didx, rows,
                  y_spm, acc):
    c = lax.axis_index("c")
    s = lax.axis_index("s")
    _zero_rows(rows, _CH, _HH)

    @pl.loop(0, _ZCH // _CH)
    def _(k):
        pltpu.sync_copy(rows, acc.at[pl.ds(s * _ZCH + k * _CH, _CH)])

    @pl.when(c == 0)
    def _():
        @pl.when(s < _NS - 1)
        def _():
            pltpu.sync_copy(yl_hbm.at[pl.ds(s * _ZCH, _ZCH)],
                            y_spm.at[pl.ds(s * _ZCH, _ZCH)])

        @pl.when(s == _NS - 1)
        def _():
            pltpu.sync_copy(yl_hbm.at[pl.ds((_NS - 1) * _ZCH, _N - (_NS - 1) * _ZCH)],
                            y_spm.at[pl.ds((_NS - 1) * _ZCH, _N - (_NS - 1) * _ZCH)])

    @pl.when(c == 1)
    def _():
        @pl.when(s < _NS - 1)
        def _():
            pltpu.sync_copy(yr_hbm.at[pl.ds(s * _ZCH, _ZCH)],
                            y_spm.at[pl.ds(s * _ZCH, _ZCH)])

        @pl.when(s == _NS - 1)
        def _():
            pltpu.sync_copy(yr_hbm.at[pl.ds((_NS - 1) * _ZCH, _N - (_NS - 1) * _ZCH)],
                            y_spm.at[pl.ds((_NS - 1) * _ZCH, _N - (_NS - 1) * _ZCH)])

    pltpu.sync_copy(src_hbm.at[s], sidx)
    pltpu.sync_copy(dst_hbm.at[s], didx)
    plsc.subcore_barrier()

    @pl.loop(0, _NCH)
    def _(j):
        pltpu.sync_copy(y_spm.at[sidx.at[j]], rows)
        pltpu.sync_copy(rows, acc.at[didx.at[j]], add=True)

    plsc.subcore_barrier()
    pltpu.sync_copy(acc.at[pl.ds(s * _ZCH, _ZCH)],
                    out_hbm.at[c, pl.ds(s * _ZCH, _ZCH)])


def _tc1_body(d0, d1, x, w, yl_out, yr_out, dinv_out):
    deg = d0[:, 0:1] + d1[:, 0:1] + 1.0
    dv = 1.0 / jnp.sqrt(deg)
    dinv_out[...] = dv
    xw = lax.dot_general(x[...], w[...], (((1,), (0,)), ((), ())), **_DOT)
    y = dv * xw
    yl_out[...] = y[:, :_HH]
    yr_out[...] = y[:, _HH:]


_tc1 = pl.pallas_call(
    _tc1_body,
    grid=(_N // _BN,),
    in_specs=[
        pl.BlockSpec((_BN, 16), lambda i: (i, 0)),
        pl.BlockSpec((_BN, 16), lambda i: (i, 0)),
        pl.BlockSpec((_BN, _FIN), lambda i: (i, 0)),
        pl.BlockSpec((_FIN, _H), lambda i: (0, 0)),
    ],
    out_specs=[
        pl.BlockSpec((_BN, _HH), lambda i: (i, 0)),
        pl.BlockSpec((_BN, _HH), lambda i: (i, 0)),
        pl.BlockSpec((_BN, 1), lambda i: (i, 0)),
    ],
    out_shape=[
        jax.ShapeDtypeStruct((_N, _HH), jnp.float32),
        jax.ShapeDtypeStruct((_N, _HH), jnp.float32),
        jax.ShapeDtypeStruct((_N, 1), jnp.float32),
    ],
)


def _tc2_body(p0, p1, yl, yr, dinv, b, w, yln, yrn):
    dv = dinv[...]
    agg = jnp.concatenate([p0[...] + yl[...], p1[...] + yr[...]], axis=1)
    h = jnp.maximum(dv * agg + b[...], 0.0)
    yn = dv * lax.dot_general(h, w[...], (((1,), (0,)), ((), ())), **_DOT)
    yln[...] = yn[:, :_HH]
    yrn[...] = yn[:, _HH:]


_tc2 = pl.pallas_call(
    _tc2_body,
    grid=(_N // _BN,),
    in_specs=[
        pl.BlockSpec((_BN, _HH), lambda i: (i, 0)),
        pl.BlockSpec((_BN, _HH), lambda i: (i, 0)),
        pl.BlockSpec((_BN, _HH), lambda i: (i, 0)),
        pl.BlockSpec((_BN, _HH), lambda i: (i, 0)),
        pl.BlockSpec((_BN, 1), lambda i: (i, 0)),
        pl.BlockSpec((1, _H), lambda i: (0, 0)),
        pl.BlockSpec((_H, _H), lambda i: (0, 0)),
    ],
    out_specs=[
        pl.BlockSpec((_BN, _HH), lambda i: (i, 0)),
        pl.BlockSpec((_BN, _HH), lambda i: (i, 0)),
    ],
    out_shape=[
        jax.ShapeDtypeStruct((_N, _HH), jnp.float32),
        jax.ShapeDtypeStruct((_N, _HH), jnp.float32),
    ],
)


def _tc3_body(p0, p1, yl, yr, dinv, b3, bat, pw1, pb1, pw2, pb2, zout,
              pooled, cnt):
    i = pl.program_id(0)

    @pl.when(i == 0)
    def _():
        pooled[...] = jnp.zeros_like(pooled)
        cnt[...] = jnp.zeros_like(cnt)

    dv = dinv[...]
    agg = jnp.concatenate([p0[...] + yl[...], p1[...] + yr[...]], axis=1)
    h3 = dv * agg + b3[...]
    gid = lax.broadcasted_iota(jnp.int32, (_BN, _G), 1)
    oh = (bat[...] == gid).astype(jnp.float32)
    pooled[...] += lax.dot_general(oh, h3, (((0,), (0,)), ((), ())), **_DOT)
    cnt[...] += lax.dot_general(oh, jnp.ones((_BN, 1), jnp.float32),
                                (((0,), (0,)), ((), ())), **_DOT)

    @pl.when(i == _N // _BN - 1)
    def _():
        hm = pooled[...] / jnp.maximum(cnt[...], 1.0)
        z = jnp.maximum(lax.dot_general(hm, pw1[...], (((1,), (0,)), ((), ())),
                                        **_DOT) + pb1[...], 0.0)
        z = lax.dot_general(z, pw2[...], (((1,), (0,)), ((), ())), **_DOT) + pb2[...]
        nrm = jnp.sqrt(jnp.sum(z * z, axis=1, keepdims=True))
        zout[...] = z / jnp.maximum(nrm, 1e-12)


_tc3 = pl.pallas_call(
    _tc3_body,
    grid=(_N // _BN,),
    in_specs=[
        pl.BlockSpec((_BN, _HH), lambda i: (i, 0)),
        pl.BlockSpec((_BN, _HH), lambda i: (i, 0)),
        pl.BlockSpec((_BN, _HH), lambda i: (i, 0)),
        pl.BlockSpec((_BN, _HH), lambda i: (i, 0)),
        pl.BlockSpec((_BN, 1), lambda i: (i, 0)),
        pl.BlockSpec((1, _H), lambda i: (0, 0)),
        pl.BlockSpec((_BN, 1), lambda i: (i, 0)),
        pl.BlockSpec((_H, _H), lambda i: (0, 0)),
        pl.BlockSpec((1, _H), lambda i: (0, 0)),
        pl.BlockSpec((_H, _OUT), lambda i: (0, 0)),
        pl.BlockSpec((1, _OUT), lambda i: (0, 0)),
    ],
    out_specs=pl.BlockSpec((_G, _OUT), lambda i: (0, 0)),
    out_shape=jax.ShapeDtypeStruct((_G, _OUT), jnp.float32),
    scratch_shapes=[
        pltpu.VMEM((_G, _H), jnp.float32),
        pltpu.VMEM((_G, 1), jnp.float32),
    ],
)


def kernel(x, edge_index, batch, W1, b1, W2, b2, W3, b3, PW1, Pb1, PW2, Pb2):
    src = edge_index[0]
    dst = edge_index[1]
    pad = _EPAD - _E
    src_p = jnp.concatenate([src, jnp.zeros((pad,), jnp.int32)]).reshape(
        _NS, _NCH, _CH)
    dst_p = jnp.concatenate([dst, jnp.full((pad,), _N, jnp.int32)]).reshape(
        _NS, _NCH, _CH)

    sc_deg = _sc_degree_kernel()
    sc_agg = _sc_aggregate_kernel()
    degp = sc_deg(dst_p)[:, :_N]
    yl, yr, dinv = _tc1(degp[0], degp[1], x, W1)
    p = sc_agg(yl, yr, src_p, dst_p)[:, :_N]
    yl, yr = _tc2(p[0], p[1], yl, yr, dinv, b1.reshape(1, _H), W2)
    p = sc_agg(yl, yr, src_p, dst_p)[:, :_N]
    yl, yr = _tc2(p[0], p[1], yl, yr, dinv, b2.reshape(1, _H), W3)
    p = sc_agg(yl, yr, src_p, dst_p)[:, :_N]
    return _tc3(p[0], p[1], yl, yr, dinv, b3.reshape(1, _H),
                batch.reshape(_N, 1), PW1, Pb1.reshape(1, _H), PW2,
                Pb2.reshape(1, _OUT))


# speedup vs baseline: 1.9218x; 1.1109x over previous
import functools

import jax
import jax.numpy as jnp
from jax import lax
from jax.experimental import pallas as pl
from jax.experimental.pallas import tpu as pltpu
from jax.experimental.pallas import tpu_sc as plsc

_N = 10000
_E = 320000
_FIN = 128
_H = 64
_HH = _H // 2
_OUT = 32
_G = 64

_NC, _NS = 2, 16
_CH = 128
_NCH = 160
_EPW = _NCH * _CH
_EPAD = _NS * _EPW
_ACC = 10240
_ZCH = _ACC // _NS
_DCH = _NCH // _NC
_BN = 2000

_DOT = dict(preferred_element_type=jnp.float32, precision=lax.Precision.HIGHEST)


def _zero_rows(buf, nrows, width):
    @pl.loop(0, nrows)
    def _(i):
        for j0 in range(0, width, 16):
            buf[i, pl.ds(j0, 16)] = jnp.zeros((16,), jnp.float32)


@functools.lru_cache(maxsize=None)
def _sc_degree_kernel():
    mesh = plsc.VectorSubcoreMesh(core_axis_name="c", subcore_axis_name="s")
    return pl.kernel(
        _sc_degree,
        out_type=jax.ShapeDtypeStruct((_NC, _ACC, 16), jnp.float32),
        mesh=mesh,
        compiler_params=pltpu.CompilerParams(use_tc_tiling_on_sc=False),
        scratch_types=[
            pltpu.VMEM((_DCH, _CH), jnp.int32),
            pltpu.VMEM((_CH, 16), jnp.float32),
            pltpu.VMEM_SHARED((_ACC, 16), jnp.float32),
        ],
    )


@functools.lru_cache(maxsize=None)
def _sc_aggregate_kernel():
    mesh = plsc.VectorSubcoreMesh(core_axis_name="c", subcore_axis_name="s")
    return pl.kernel(
        _sc_aggregate,
        out_type=jax.ShapeDtypeStruct((_NC, _ACC, _HH), jnp.float32),
        mesh=mesh,
        compiler_params=pltpu.CompilerParams(use_tc_tiling_on_sc=False),
        scratch_types=[
            pltpu.VMEM((_NCH, _CH), jnp.int32),
            pltpu.VMEM((_NCH, _CH), jnp.int32),
            pltpu.VMEM((_CH, _HH), jnp.float32),
            pltpu.VMEM((_CH, _HH), jnp.float32),
            pltpu.VMEM_SHARED((_N, _HH), jnp.float32),
            pltpu.VMEM_SHARED((_ACC, _HH), jnp.float32),
            pltpu.SemaphoreType.DMA,
            pltpu.SemaphoreType.DMA,
        ],
    )


def _sc_degree(dst_hbm, out_hbm, didx, rbuf, acc):
    c = lax.axis_index("c")
    s = lax.axis_index("s")
    _zero_rows(rbuf, _CH, 16)

    @pl.loop(0, _ZCH // _CH)
    def _(k):
        pltpu.sync_copy(rbuf, acc.at[pl.ds(s * _ZCH + k * _CH, _CH)])

    lanes = lax.iota(jnp.int32, 16)
    one0 = jnp.where(lanes == 0, jnp.float32(1.0), jnp.float32(0.0))

    @pl.loop(0, _CH)
    def _(i):
        rbuf[i, pl.ds(0, 16)] = one0

    pltpu.sync_copy(dst_hbm.at[s, pl.ds(c * _DCH, _DCH)], didx)
    plsc.subcore_barrier()

    @pl.loop(0, _DCH)
    def _(j):
        pltpu.sync_copy(rbuf, acc.at[didx.at[j]], add=True)

    plsc.subcore_barrier()
    pltpu.sync_copy(acc.at[pl.ds(s * _ZCH, _ZCH)],
                    out_hbm.at[c, pl.ds(s * _ZCH, _ZCH)])


def _sc_aggregate(yl_hbm, yr_hbm, src_hbm, dst_hbm, out_hbm, sidx, didx,
                  rows_a, rows_b, y_spm, acc, sem_a, sem_b):
    c = lax.axis_index("c")
    s = lax.axis_index("s")
    _zero_rows(rows_a, _CH, _HH)

    @pl.loop(0, _ZCH // _CH)
    def _(k):
        pltpu.sync_copy(rows_a, acc.at[pl.ds(s * _ZCH + k * _CH, _CH)])

    @pl.when(c == 0)
    def _():
        @pl.when(s < _NS - 1)
        def _():
            pltpu.sync_copy(yl_hbm.at[pl.ds(s * _ZCH, _ZCH)],
                            y_spm.at[pl.ds(s * _ZCH, _ZCH)])

        @pl.when(s == _NS - 1)
        def _():
            pltpu.sync_copy(yl_hbm.at[pl.ds((_NS - 1) * _ZCH, _N - (_NS - 1) * _ZCH)],
                            y_spm.at[pl.ds((_NS - 1) * _ZCH, _N - (_NS - 1) * _ZCH)])

    @pl.when(c == 1)
    def _():
        @pl.when(s < _NS - 1)
        def _():
            pltpu.sync_copy(yr_hbm.at[pl.ds(s * _ZCH, _ZCH)],
                            y_spm.at[pl.ds(s * _ZCH, _ZCH)])

        @pl.when(s == _NS - 1)
        def _():
            pltpu.sync_copy(yr_hbm.at[pl.ds((_NS - 1) * _ZCH, _N - (_NS - 1) * _ZCH)],
                            y_spm.at[pl.ds((_NS - 1) * _ZCH, _N - (_NS - 1) * _ZCH)])

    pltpu.sync_copy(src_hbm.at[s], sidx)
    pltpu.sync_copy(dst_hbm.at[s], didx)
    plsc.subcore_barrier()

    @pl.loop(0, _NCH, step=2)
    def _(j):
        cp_a = pltpu.async_copy(y_spm.at[sidx.at[j]], rows_a, sem_a)
        cp_b = pltpu.async_copy(y_spm.at[sidx.at[j + 1]], rows_b, sem_b)
        cp_a.wait()
        pltpu.sync_copy(rows_a, acc.at[didx.at[j]], add=True)
        cp_b.wait()
        pltpu.sync_copy(rows_b, acc.at[didx.at[j + 1]], add=True)

    plsc.subcore_barrier()
    pltpu.sync_copy(acc.at[pl.ds(s * _ZCH, _ZCH)],
                    out_hbm.at[c, pl.ds(s * _ZCH, _ZCH)])


def _tc1_body(d, x, w, yl_out, yr_out, dinv_out):
    deg = d[0, :, 0:1] + d[1, :, 0:1] + 1.0
    dv = 1.0 / jnp.sqrt(deg)
    dinv_out[...] = dv
    xw = lax.dot_general(x[...], w[...], (((1,), (0,)), ((), ())), **_DOT)
    y = dv * xw
    yl_out[...] = y[:, :_HH]
    yr_out[...] = y[:, _HH:]


_tc1 = pl.pallas_call(
    _tc1_body,
    grid=(_N // _BN,),
    in_specs=[
        pl.BlockSpec((2, _BN, 16), lambda i: (0, i, 0)),
        pl.BlockSpec((_BN, _FIN), lambda i: (i, 0)),
        pl.BlockSpec((_FIN, _H), lambda i: (0, 0)),
    ],
    out_specs=[
        pl.BlockSpec((_BN, _HH), lambda i: (i, 0)),
        pl.BlockSpec((_BN, _HH), lambda i: (i, 0)),
        pl.BlockSpec((_BN, 1), lambda i: (i, 0)),
    ],
    out_shape=[
        jax.ShapeDtypeStruct((_N, _HH), jnp.float32),
        jax.ShapeDtypeStruct((_N, _HH), jnp.float32),
        jax.ShapeDtypeStruct((_N, 1), jnp.float32),
    ],
)


def _tc2_body(p, yl, yr, dinv, b, w, yln, yrn):
    dv = dinv[...]
    agg = jnp.concatenate([p[0] + yl[...], p[1] + yr[...]], axis=1)
    h = jnp.maximum(dv * agg + b[...], 0.0)
    yn = dv * lax.dot_general(h, w[...], (((1,), (0,)), ((), ())), **_DOT)
    yln[...] = yn[:, :_HH]
    yrn[...] = yn[:, _HH:]


_tc2 = pl.pallas_call(
    _tc2_body,
    grid=(_N // _BN,),
    in_specs=[
        pl.BlockSpec((2, _BN, _HH), lambda i: (0, i, 0)),
        pl.BlockSpec((_BN, _HH), lambda i: (i, 0)),
        pl.BlockSpec((_BN, _HH), lambda i: (i, 0)),
        pl.BlockSpec((_BN, 1), lambda i: (i, 0)),
        pl.BlockSpec((1, _H), lambda i: (0, 0)),
        pl.BlockSpec((_H, _H), lambda i: (0, 0)),
    ],
    out_specs=[
        pl.BlockSpec((_BN, _HH), lambda i: (i, 0)),
        pl.BlockSpec((_BN, _HH), lambda i: (i, 0)),
    ],
    out_shape=[
        jax.ShapeDtypeStruct((_N, _HH), jnp.float32),
        jax.ShapeDtypeStruct((_N, _HH), jnp.float32),
    ],
)


def _tc3_body(p, yl, yr, dinv, b3, bat, pw1, pb1, pw2, pb2, zout,
              pooled, cnt):
    i = pl.program_id(0)

    @pl.when(i == 0)
    def _():
        pooled[...] = jnp.zeros_like(pooled)
        cnt[...] = jnp.zeros_like(cnt)

    dv = dinv[...]
    agg = jnp.concatenate([p[0] + yl[...], p[1] + yr[...]], axis=1)
    h3 = dv * agg + b3[...]
    gid = lax.broadcasted_iota(jnp.int32, (_BN, _G), 1)
    oh = (bat[...] == gid).astype(jnp.float32)
    pooled[...] += lax.dot_general(oh, h3, (((0,), (0,)), ((), ())), **_DOT)
    cnt[...] += lax.dot_general(oh, jnp.ones((_BN, 1), jnp.float32),
                                (((0,), (0,)), ((), ())), **_DOT)

    @pl.when(i == _N // _BN - 1)
    def _():
        hm = pooled[...] / jnp.maximum(cnt[...], 1.0)
        z = jnp.maximum(lax.dot_general(hm, pw1[...], (((1,), (0,)), ((), ())),
                                        **_DOT) + pb1[...], 0.0)
        z = lax.dot_general(z, pw2[...], (((1,), (0,)), ((), ())), **_DOT) + pb2[...]
        nrm = jnp.sqrt(jnp.sum(z * z, axis=1, keepdims=True))
        zout[...] = z / jnp.maximum(nrm, 1e-12)


_tc3 = pl.pallas_call(
    _tc3_body,
    grid=(_N // _BN,),
    in_specs=[
        pl.BlockSpec((2, _BN, _HH), lambda i: (0, i, 0)),
        pl.BlockSpec((_BN, _HH), lambda i: (i, 0)),
        pl.BlockSpec((_BN, _HH), lambda i: (i, 0)),
        pl.BlockSpec((_BN, 1), lambda i: (i, 0)),
        pl.BlockSpec((1, _H), lambda i: (0, 0)),
        pl.BlockSpec((_BN, 1), lambda i: (i, 0)),
        pl.BlockSpec((_H, _H), lambda i: (0, 0)),
        pl.BlockSpec((1, _H), lambda i: (0, 0)),
        pl.BlockSpec((_H, _OUT), lambda i: (0, 0)),
        pl.BlockSpec((1, _OUT), lambda i: (0, 0)),
    ],
    out_specs=pl.BlockSpec((_G, _OUT), lambda i: (0, 0)),
    out_shape=jax.ShapeDtypeStruct((_G, _OUT), jnp.float32),
    scratch_shapes=[
        pltpu.VMEM((_G, _H), jnp.float32),
        pltpu.VMEM((_G, 1), jnp.float32),
    ],
)


def kernel(x, edge_index, batch, W1, b1, W2, b2, W3, b3, PW1, Pb1, PW2, Pb2):
    src = edge_index[0]
    dst = edge_index[1]
    pad = _EPAD - _E
    src_p = jnp.concatenate([src, jnp.zeros((pad,), jnp.int32)]).reshape(
        _NS, _NCH, _CH)
    dst_p = jnp.concatenate([dst, jnp.full((pad,), _N, jnp.int32)]).reshape(
        _NS, _NCH, _CH)

    sc_deg = _sc_degree_kernel()
    sc_agg = _sc_aggregate_kernel()
    degp = sc_deg(dst_p)
    yl, yr, dinv = _tc1(degp, x, W1)
    p = sc_agg(yl, yr, src_p, dst_p)
    yl, yr = _tc2(p, yl, yr, dinv, b1.reshape(1, _H), W2)
    p = sc_agg(yl, yr, src_p, dst_p)
    yl, yr = _tc2(p, yl, yr, dinv, b2.reshape(1, _H), W3)
    p = sc_agg(yl, yr, src_p, dst_p)
    return _tc3(p, yl, yr, dinv, b3.reshape(1, _H),
                batch.reshape(_N, 1), PW1, Pb1.reshape(1, _H), PW2,
                Pb2.reshape(1, _OUT))


# speedup vs baseline: 1.9775x; 1.0290x over previous
import functools

import jax
import jax.numpy as jnp
from jax import lax
from jax.experimental import pallas as pl
from jax.experimental.pallas import tpu as pltpu
from jax.experimental.pallas import tpu_sc as plsc

_N = 10000
_E = 320000
_FIN = 128
_H = 64
_HH = _H // 2
_OUT = 32
_G = 64

_NC, _NS = 2, 16
_CH = 128
_NCH = 160
_EPW = _NCH * _CH
_EPAD = _NS * _EPW
_ACC = 10240
_ZCH = _ACC // _NS
_DCH = _NCH // _NC
_BN = 2000

_DOT = dict(preferred_element_type=jnp.float32, precision=lax.Precision.HIGHEST)


def _zero_rows(buf, nrows, width):
    @pl.loop(0, nrows)
    def _(i):
        for j0 in range(0, width, 16):
            buf[i, pl.ds(j0, 16)] = jnp.zeros((16,), jnp.float32)


@functools.lru_cache(maxsize=None)
def _sc_degree_kernel():
    mesh = plsc.VectorSubcoreMesh(core_axis_name="c", subcore_axis_name="s")
    return pl.kernel(
        _sc_degree,
        out_type=jax.ShapeDtypeStruct((_NC, _ACC, 16), jnp.float32),
        mesh=mesh,
        compiler_params=pltpu.CompilerParams(use_tc_tiling_on_sc=False),
        scratch_types=[
            pltpu.VMEM((_DCH, _CH), jnp.int32),
            pltpu.VMEM((_CH, 16), jnp.float32),
            pltpu.VMEM_SHARED((_ACC, 16), jnp.float32),
        ],
    )


@functools.lru_cache(maxsize=None)
def _sc_aggregate_kernel():
    mesh = plsc.VectorSubcoreMesh(core_axis_name="c", subcore_axis_name="s")
    return pl.kernel(
        _sc_aggregate,
        out_type=jax.ShapeDtypeStruct((_NC, _ACC, _HH), jnp.float32),
        mesh=mesh,
        compiler_params=pltpu.CompilerParams(use_tc_tiling_on_sc=False),
        scratch_types=[
            pltpu.VMEM((_NCH, _CH), jnp.int32),
            pltpu.VMEM((_NCH, _CH), jnp.int32),
            pltpu.VMEM((_CH, _HH), jnp.float32),
            pltpu.VMEM((_CH, _HH), jnp.float32),
            pltpu.VMEM_SHARED((_N, _HH), jnp.float32),
            pltpu.VMEM_SHARED((_ACC, _HH), jnp.float32),
            pltpu.SemaphoreType.DMA,
            pltpu.SemaphoreType.DMA,
        ],
    )


def _sc_degree(dst_hbm, out_hbm, didx, rbuf, acc):
    c = lax.axis_index("c")
    s = lax.axis_index("s")
    _zero_rows(rbuf, _CH, 16)

    @pl.loop(0, _ZCH // _CH)
    def _(k):
        pltpu.sync_copy(rbuf, acc.at[pl.ds(s * _ZCH + k * _CH, _CH)])

    lanes = lax.iota(jnp.int32, 16)
    one0 = jnp.where(lanes == 0, jnp.float32(1.0), jnp.float32(0.0))

    @pl.loop(0, _CH)
    def _(i):
        rbuf[i, pl.ds(0, 16)] = one0

    pltpu.sync_copy(dst_hbm.at[s, pl.ds(c * _DCH, _DCH)], didx)
    plsc.subcore_barrier()

    @pl.loop(0, _DCH)
    def _(j):
        pltpu.sync_copy(rbuf, acc.at[didx.at[j]], add=True)

    plsc.subcore_barrier()
    pltpu.sync_copy(acc.at[pl.ds(s * _ZCH, _ZCH)],
                    out_hbm.at[c, pl.ds(s * _ZCH, _ZCH)])


def _sc_aggregate(yl_hbm, yr_hbm, src_hbm, dst_hbm, out_hbm, sidx, didx,
                  rows_a, rows_b, y_spm, acc, sem_a, sem_b):
    c = lax.axis_index("c")
    s = lax.axis_index("s")
    _zero_rows(rows_a, _CH, _HH)

    @pl.loop(0, _ZCH // _CH)
    def _(k):
        pltpu.sync_copy(rows_a, acc.at[pl.ds(s * _ZCH + k * _CH, _CH)])

    @pl.when(c == 0)
    def _():
        @pl.when(s < _NS - 1)
        def _():
            pltpu.sync_copy(yl_hbm.at[pl.ds(s * _ZCH, _ZCH)],
                            y_spm.at[pl.ds(s * _ZCH, _ZCH)])

        @pl.when(s == _NS - 1)
        def _():
            pltpu.sync_copy(yl_hbm.at[pl.ds((_NS - 1) * _ZCH, _N - (_NS - 1) * _ZCH)],
                            y_spm.at[pl.ds((_NS - 1) * _ZCH, _N - (_NS - 1) * _ZCH)])

    @pl.when(c == 1)
    def _():
        @pl.when(s < _NS - 1)
        def _():
            pltpu.sync_copy(yr_hbm.at[pl.ds(s * _ZCH, _ZCH)],
                            y_spm.at[pl.ds(s * _ZCH, _ZCH)])

        @pl.when(s == _NS - 1)
        def _():
            pltpu.sync_copy(yr_hbm.at[pl.ds((_NS - 1) * _ZCH, _N - (_NS - 1) * _ZCH)],
                            y_spm.at[pl.ds((_NS - 1) * _ZCH, _N - (_NS - 1) * _ZCH)])

    pltpu.sync_copy(src_hbm.at[s], sidx)
    pltpu.sync_copy(dst_hbm.at[s], didx)
    plsc.subcore_barrier()

    @pl.loop(0, _NCH)
    def _(j):
        pltpu.sync_copy(y_spm.at[sidx.at[j]], rows_a, )
        pltpu.sync_copy(rows_a, acc.at[didx.at[j]], add=True)

    plsc.subcore_barrier()
    pltpu.sync_copy(acc.at[pl.ds(s * _ZCH, _ZCH)],
                    out_hbm.at[c, pl.ds(s * _ZCH, _ZCH)])


def _tc1_body(d, x, w, yl_out, yr_out, dinv_out):
    deg = d[0, :, 0:1] + d[1, :, 0:1] + 1.0
    dv = 1.0 / jnp.sqrt(deg)
    dinv_out[...] = dv
    xw = lax.dot_general(x[...], w[...], (((1,), (0,)), ((), ())), **_DOT)
    y = dv * xw
    yl_out[...] = y[:, :_HH]
    yr_out[...] = y[:, _HH:]


_tc1 = pl.pallas_call(
    _tc1_body,
    grid=(_N // _BN,),
    in_specs=[
        pl.BlockSpec((2, _BN, 16), lambda i: (0, i, 0)),
        pl.BlockSpec((_BN, _FIN), lambda i: (i, 0)),
        pl.BlockSpec((_FIN, _H), lambda i: (0, 0)),
    ],
    out_specs=[
        pl.BlockSpec((_BN, _HH), lambda i: (i, 0)),
        pl.BlockSpec((_BN, _HH), lambda i: (i, 0)),
        pl.BlockSpec((_BN, 1), lambda i: (i, 0)),
    ],
    out_shape=[
        jax.ShapeDtypeStruct((_N, _HH), jnp.float32),
        jax.ShapeDtypeStruct((_N, _HH), jnp.float32),
        jax.ShapeDtypeStruct((_N, 1), jnp.float32),
    ],
)


def _tc2_body(p, yl, yr, dinv, b, w, yln, yrn):
    dv = dinv[...]
    agg = jnp.concatenate([p[0] + yl[...], p[1] + yr[...]], axis=1)
    h = jnp.maximum(dv * agg + b[...], 0.0)
    yn = dv * lax.dot_general(h, w[...], (((1,), (0,)), ((), ())), **_DOT)
    yln[...] = yn[:, :_HH]
    yrn[...] = yn[:, _HH:]


_tc2 = pl.pallas_call(
    _tc2_body,
    grid=(_N // _BN,),
    in_specs=[
        pl.BlockSpec((2, _BN, _HH), lambda i: (0, i, 0)),
        pl.BlockSpec((_BN, _HH), lambda i: (i, 0)),
        pl.BlockSpec((_BN, _HH), lambda i: (i, 0)),
        pl.BlockSpec((_BN, 1), lambda i: (i, 0)),
        pl.BlockSpec((1, _H), lambda i: (0, 0)),
        pl.BlockSpec((_H, _H), lambda i: (0, 0)),
    ],
    out_specs=[
        pl.BlockSpec((_BN, _HH), lambda i: (i, 0)),
        pl.BlockSpec((_BN, _HH), lambda i: (i, 0)),
    ],
    out_shape=[
        jax.ShapeDtypeStruct((_N, _HH), jnp.float32),
        jax.ShapeDtypeStruct((_N, _HH), jnp.float32),
    ],
)


def _tc3_body(p, yl, yr, dinv, b3, bat, pw1, pb1, pw2, pb2, zout,
              pooled, cnt):
    i = pl.program_id(0)

    @pl.when(i == 0)
    def _():
        pooled[...] = jnp.zeros_like(pooled)
        cnt[...] = jnp.zeros_like(cnt)

    dv = dinv[...]
    agg = jnp.concatenate([p[0] + yl[...], p[1] + yr[...]], axis=1)
    h3 = dv * agg + b3[...]
    gid = lax.broadcasted_iota(jnp.int32, (_BN, _G), 1)
    oh = (bat[...] == gid).astype(jnp.float32)
    pooled[...] += lax.dot_general(oh, h3, (((0,), (0,)), ((), ())), **_DOT)
    cnt[...] += lax.dot_general(oh, jnp.ones((_BN, 1), jnp.float32),
                                (((0,), (0,)), ((), ())), **_DOT)

    @pl.when(i == _N // _BN - 1)
    def _():
        hm = pooled[...] / jnp.maximum(cnt[...], 1.0)
        z = jnp.maximum(lax.dot_general(hm, pw1[...], (((1,), (0,)), ((), ())),
                                        **_DOT) + pb1[...], 0.0)
        z = lax.dot_general(z, pw2[...], (((1,), (0,)), ((), ())), **_DOT) + pb2[...]
        nrm = jnp.sqrt(jnp.sum(z * z, axis=1, keepdims=True))
        zout[...] = z / jnp.maximum(nrm, 1e-12)


_tc3 = pl.pallas_call(
    _tc3_body,
    grid=(_N // _BN,),
    in_specs=[
        pl.BlockSpec((2, _BN, _HH), lambda i: (0, i, 0)),
        pl.BlockSpec((_BN, _HH), lambda i: (i, 0)),
        pl.BlockSpec((_BN, _HH), lambda i: (i, 0)),
        pl.BlockSpec((_BN, 1), lambda i: (i, 0)),
        pl.BlockSpec((1, _H), lambda i: (0, 0)),
        pl.BlockSpec((_BN, 1), lambda i: (i, 0)),
        pl.BlockSpec((_H, _H), lambda i: (0, 0)),
        pl.BlockSpec((1, _H), lambda i: (0, 0)),
        pl.BlockSpec((_H, _OUT), lambda i: (0, 0)),
        pl.BlockSpec((1, _OUT), lambda i: (0, 0)),
    ],
    out_specs=pl.BlockSpec((_G, _OUT), lambda i: (0, 0)),
    out_shape=jax.ShapeDtypeStruct((_G, _OUT), jnp.float32),
    scratch_shapes=[
        pltpu.VMEM((_G, _H), jnp.float32),
        pltpu.VMEM((_G, 1), jnp.float32),
    ],
)


def kernel(x, edge_index, batch, W1, b1, W2, b2, W3, b3, PW1, Pb1, PW2, Pb2):
    src = edge_index[0]
    dst = edge_index[1]
    pad = _EPAD - _E
    src_p = jnp.concatenate([src, jnp.zeros((pad,), jnp.int32)]).reshape(
        _NS, _NCH, _CH)
    dst_p = jnp.concatenate([dst, jnp.full((pad,), _N, jnp.int32)]).reshape(
        _NS, _NCH, _CH)

    sc_deg = _sc_degree_kernel()
    sc_agg = _sc_aggregate_kernel()
    degp = sc_deg(dst_p)
    yl, yr, dinv = _tc1(degp, x, W1)
    p = sc_agg(yl, yr, src_p, dst_p)
    yl, yr = _tc2(p, yl, yr, dinv, b1.reshape(1, _H), W2)
    p = sc_agg(yl, yr, src_p, dst_p)
    yl, yr = _tc2(p, yl, yr, dinv, b2.reshape(1, _H), W3)
    p = sc_agg(yl, yr, src_p, dst_p)
    return _tc3(p, yl, yr, dinv, b3.reshape(1, _H),
                batch.reshape(_N, 1), PW1, Pb1.reshape(1, _H), PW2,
                Pb2.reshape(1, _OUT))


# speedup vs baseline: 2.0045x; 1.0136x over previous
import functools

import jax
import jax.numpy as jnp
from jax import lax
from jax.experimental import pallas as pl
from jax.experimental.pallas import tpu as pltpu
from jax.experimental.pallas import tpu_sc as plsc

_N = 10000
_E = 320000
_FIN = 128
_H = 64
_HH = _H // 2
_OUT = 32
_G = 64

_NC, _NS = 2, 16
_CH = 128
_NCH = 160
_EPW = _NCH * _CH
_EPAD = _NS * _EPW
_ACC = 10240
_ZCH = _ACC // _NS
_DCH = _NCH // _NC
_BN = 2000

_DOT = dict(preferred_element_type=jnp.float32)


def _zero_rows(buf, nrows, width):
    @pl.loop(0, nrows)
    def _(i):
        for j0 in range(0, width, 16):
            buf[i, pl.ds(j0, 16)] = jnp.zeros((16,), jnp.float32)


@functools.lru_cache(maxsize=None)
def _sc_degree_kernel():
    mesh = plsc.VectorSubcoreMesh(core_axis_name="c", subcore_axis_name="s")
    return pl.kernel(
        _sc_degree,
        out_type=jax.ShapeDtypeStruct((_NC, _ACC, 16), jnp.float32),
        mesh=mesh,
        compiler_params=pltpu.CompilerParams(use_tc_tiling_on_sc=False),
        scratch_types=[
            pltpu.VMEM((_DCH, _CH), jnp.int32),
            pltpu.VMEM((_CH, 16), jnp.float32),
            pltpu.VMEM_SHARED((_ACC, 16), jnp.float32),
        ],
    )


@functools.lru_cache(maxsize=None)
def _sc_aggregate_kernel():
    mesh = plsc.VectorSubcoreMesh(core_axis_name="c", subcore_axis_name="s")
    return pl.kernel(
        _sc_aggregate,
        out_type=jax.ShapeDtypeStruct((_NC, _ACC, _HH), jnp.float32),
        mesh=mesh,
        compiler_params=pltpu.CompilerParams(use_tc_tiling_on_sc=False),
        scratch_types=[
            pltpu.VMEM((_NCH, _CH), jnp.int32),
            pltpu.VMEM((_NCH, _CH), jnp.int32),
            pltpu.VMEM((_CH, _HH), jnp.float32),
            pltpu.VMEM((_CH, _HH), jnp.float32),
            pltpu.VMEM_SHARED((_N, _HH), jnp.float32),
            pltpu.VMEM_SHARED((_ACC, _HH), jnp.float32),
            pltpu.SemaphoreType.DMA,
            pltpu.SemaphoreType.DMA,
        ],
    )


def _sc_degree(dst_hbm, out_hbm, didx, rbuf, acc):
    c = lax.axis_index("c")
    s = lax.axis_index("s")
    _zero_rows(rbuf, _CH, 16)

    @pl.loop(0, _ZCH // _CH)
    def _(k):
        pltpu.sync_copy(rbuf, acc.at[pl.ds(s * _ZCH + k * _CH, _CH)])

    lanes = lax.iota(jnp.int32, 16)
    one0 = jnp.where(lanes == 0, jnp.float32(1.0), jnp.float32(0.0))

    @pl.loop(0, _CH)
    def _(i):
        rbuf[i, pl.ds(0, 16)] = one0

    pltpu.sync_copy(dst_hbm.at[s, pl.ds(c * _DCH, _DCH)], didx)
    plsc.subcore_barrier()

    @pl.loop(0, _DCH)
    def _(j):
        pltpu.sync_copy(rbuf, acc.at[didx.at[j]], add=True)

    plsc.subcore_barrier()
    pltpu.sync_copy(acc.at[pl.ds(s * _ZCH, _ZCH)],
                    out_hbm.at[c, pl.ds(s * _ZCH, _ZCH)])


def _sc_aggregate(yl_hbm, yr_hbm, src_hbm, dst_hbm, out_hbm, sidx, didx,
                  rows_a, rows_b, y_spm, acc, sem_a, sem_b):
    c = lax.axis_index("c")
    s = lax.axis_index("s")
    _zero_rows(rows_a, _CH, _HH)

    @pl.loop(0, _ZCH // _CH)
    def _(k):
        pltpu.sync_copy(rows_a, acc.at[pl.ds(s * _ZCH + k * _CH, _CH)])

    @pl.when(c == 0)
    def _():
        @pl.when(s < _NS - 1)
        def _():
            pltpu.sync_copy(yl_hbm.at[pl.ds(s * _ZCH, _ZCH)],
                            y_spm.at[pl.ds(s * _ZCH, _ZCH)])

        @pl.when(s == _NS - 1)
        def _():
            pltpu.sync_copy(yl_hbm.at[pl.ds((_NS - 1) * _ZCH, _N - (_NS - 1) * _ZCH)],
                            y_spm.at[pl.ds((_NS - 1) * _ZCH, _N - (_NS - 1) * _ZCH)])

    @pl.when(c == 1)
    def _():
        @pl.when(s < _NS - 1)
        def _():
            pltpu.sync_copy(yr_hbm.at[pl.ds(s * _ZCH, _ZCH)],
                            y_spm.at[pl.ds(s * _ZCH, _ZCH)])

        @pl.when(s == _NS - 1)
        def _():
            pltpu.sync_copy(yr_hbm.at[pl.ds((_NS - 1) * _ZCH, _N - (_NS - 1) * _ZCH)],
                            y_spm.at[pl.ds((_NS - 1) * _ZCH, _N - (_NS - 1) * _ZCH)])

    pltpu.sync_copy(src_hbm.at[s], sidx)
    pltpu.sync_copy(dst_hbm.at[s], didx)
    plsc.subcore_barrier()

    @pl.loop(0, _NCH)
    def _(j):
        pltpu.sync_copy(y_spm.at[sidx.at[j]], rows_a, )
        pltpu.sync_copy(rows_a, acc.at[didx.at[j]], add=True)

    plsc.subcore_barrier()
    pltpu.sync_copy(acc.at[pl.ds(s * _ZCH, _ZCH)],
                    out_hbm.at[c, pl.ds(s * _ZCH, _ZCH)])


def _tcxw_body(x, w, xw_out):
    xw_out[...] = lax.dot_general(x[...], w[...], (((1,), (0,)), ((), ())),
                                  **_DOT)


_tcxw = pl.pallas_call(
    _tcxw_body,
    grid=(_N // _BN,),
    in_specs=[
        pl.BlockSpec((_BN, _FIN), lambda i: (i, 0)),
        pl.BlockSpec((_FIN, _H), lambda i: (0, 0)),
    ],
    out_specs=pl.BlockSpec((_BN, _H), lambda i: (i, 0)),
    out_shape=jax.ShapeDtypeStruct((_N, _H), jnp.float32),
)


def _tc1_body(d, xw, yl_out, yr_out, dinv_out):
    deg = d[0, :, 0:1] + d[1, :, 0:1] + 1.0
    dv = 1.0 / jnp.sqrt(deg)
    dinv_out[...] = dv
    y = dv * xw[...]
    yl_out[...] = y[:, :_HH]
    yr_out[...] = y[:, _HH:]


_tc1 = pl.pallas_call(
    _tc1_body,
    grid=(_N // _BN,),
    in_specs=[
        pl.BlockSpec((2, _BN, 16), lambda i: (0, i, 0)),
        pl.BlockSpec((_BN, _H), lambda i: (i, 0)),
    ],
    out_specs=[
        pl.BlockSpec((_BN, _HH), lambda i: (i, 0)),
        pl.BlockSpec((_BN, _HH), lambda i: (i, 0)),
        pl.BlockSpec((_BN, 1), lambda i: (i, 0)),
    ],
    out_shape=[
        jax.ShapeDtypeStruct((_N, _HH), jnp.float32),
        jax.ShapeDtypeStruct((_N, _HH), jnp.float32),
        jax.ShapeDtypeStruct((_N, 1), jnp.float32),
    ],
)


def _tc2_body(p, yl, yr, dinv, b, w, yln, yrn):
    dv = dinv[...]
    agg = jnp.concatenate([p[0] + yl[...], p[1] + yr[...]], axis=1)
    h = jnp.maximum(dv * agg + b[...], 0.0)
    yn = dv * lax.dot_general(h, w[...], (((1,), (0,)), ((), ())), **_DOT)
    yln[...] = yn[:, :_HH]
    yrn[...] = yn[:, _HH:]


_tc2 = pl.pallas_call(
    _tc2_body,
    grid=(_N // _BN,),
    in_specs=[
        pl.BlockSpec((2, _BN, _HH), lambda i: (0, i, 0)),
        pl.BlockSpec((_BN, _HH), lambda i: (i, 0)),
        pl.BlockSpec((_BN, _HH), lambda i: (i, 0)),
        pl.BlockSpec((_BN, 1), lambda i: (i, 0)),
        pl.BlockSpec((1, _H), lambda i: (0, 0)),
        pl.BlockSpec((_H, _H), lambda i: (0, 0)),
    ],
    out_specs=[
        pl.BlockSpec((_BN, _HH), lambda i: (i, 0)),
        pl.BlockSpec((_BN, _HH), lambda i: (i, 0)),
    ],
    out_shape=[
        jax.ShapeDtypeStruct((_N, _HH), jnp.float32),
        jax.ShapeDtypeStruct((_N, _HH), jnp.float32),
    ],
)


def _tc3_body(p, yl, yr, dinv, b3, bat, pw1, pb1, pw2, pb2, zout,
              pooled, cnt):
    i = pl.program_id(0)

    @pl.when(i == 0)
    def _():
        pooled[...] = jnp.zeros_like(pooled)
        cnt[...] = jnp.zeros_like(cnt)

    dv = dinv[...]
    agg = jnp.concatenate([p[0] + yl[...], p[1] + yr[...]], axis=1)
    h3 = dv * agg + b3[...]
    gid = lax.broadcasted_iota(jnp.int32, (_BN, _G), 1)
    oh = (bat[...] == gid).astype(jnp.float32)
    pooled[...] += lax.dot_general(oh, h3, (((0,), (0,)), ((), ())), **_DOT)
    cnt[...] += lax.dot_general(oh, jnp.ones((_BN, 1), jnp.float32),
                                (((0,), (0,)), ((), ())), **_DOT)

    @pl.when(i == _N // _BN - 1)
    def _():
        hm = pooled[...] / jnp.maximum(cnt[...], 1.0)
        z = jnp.maximum(lax.dot_general(hm, pw1[...], (((1,), (0,)), ((), ())),
                                        **_DOT) + pb1[...], 0.0)
        z = lax.dot_general(z, pw2[...], (((1,), (0,)), ((), ())), **_DOT) + pb2[...]
        nrm = jnp.sqrt(jnp.sum(z * z, axis=1, keepdims=True))
        zout[...] = z / jnp.maximum(nrm, 1e-12)


_tc3 = pl.pallas_call(
    _tc3_body,
    grid=(_N // _BN,),
    in_specs=[
        pl.BlockSpec((2, _BN, _HH), lambda i: (0, i, 0)),
        pl.BlockSpec((_BN, _HH), lambda i: (i, 0)),
        pl.BlockSpec((_BN, _HH), lambda i: (i, 0)),
        pl.BlockSpec((_BN, 1), lambda i: (i, 0)),
        pl.BlockSpec((1, _H), lambda i: (0, 0)),
        pl.BlockSpec((_BN, 1), lambda i: (i, 0)),
        pl.BlockSpec((_H, _H), lambda i: (0, 0)),
        pl.BlockSpec((1, _H), lambda i: (0, 0)),
        pl.BlockSpec((_H, _OUT), lambda i: (0, 0)),
        pl.BlockSpec((1, _OUT), lambda i: (0, 0)),
    ],
    out_specs=pl.BlockSpec((_G, _OUT), lambda i: (0, 0)),
    out_shape=jax.ShapeDtypeStruct((_G, _OUT), jnp.float32),
    scratch_shapes=[
        pltpu.VMEM((_G, _H), jnp.float32),
        pltpu.VMEM((_G, 1), jnp.float32),
    ],
)


def kernel(x, edge_index, batch, W1, b1, W2, b2, W3, b3, PW1, Pb1, PW2, Pb2):
    src = edge_index[0]
    dst = edge_index[1]
    pad = _EPAD - _E
    src_p = jnp.concatenate([src, jnp.zeros((pad,), jnp.int32)]).reshape(
        _NS, _NCH, _CH)
    dst_p = jnp.concatenate([dst, jnp.full((pad,), _N, jnp.int32)]).reshape(
        _NS, _NCH, _CH)

    sc_deg = _sc_degree_kernel()
    sc_agg = _sc_aggregate_kernel()
    xw1 = _tcxw(x, W1)
    degp = sc_deg(dst_p)
    yl, yr, dinv = _tc1(degp, xw1)
    p = sc_agg(yl, yr, src_p, dst_p)
    yl, yr = _tc2(p, yl, yr, dinv, b1.reshape(1, _H), W2)
    p = sc_agg(yl, yr, src_p, dst_p)
    yl, yr = _tc2(p, yl, yr, dinv, b2.reshape(1, _H), W3)
    p = sc_agg(yl, yr, src_p, dst_p)
    return _tc3(p, yl, yr, dinv, b3.reshape(1, _H),
                batch.reshape(_N, 1), PW1, Pb1.reshape(1, _H), PW2,
                Pb2.reshape(1, _OUT))
